# baseline (speedup 1.0000x reference)
import functools

import jax
import jax.numpy as jnp
from jax import lax
from jax.experimental import pallas as pl
from jax.experimental.pallas import tpu as pltpu
from jax.experimental.pallas import tpu_sc as plsc

NC = 2
NS = 16
NW = NC * NS
B = 16384
D = 64
L = 16
BPW = B // NW
CH = 128
NCH = BPW // CH
BK = 512

_mesh = plsc.VectorSubcoreMesh(core_axis_name="c", subcore_axis_name="s")

_GATHER_DN = lax.GatherDimensionNumbers(
    offset_dims=(), collapsed_slice_dims=(0,), start_index_map=(0,))


def _perm(v, idx):
    return lax.gather(v, idx[:, None], _GATHER_DN, slice_sizes=(1,),
                      mode=lax.GatherScatterMode.PROMISE_IN_BOUNDS)


def _pack_rows(table):
    v, d = table.shape
    grid = (v + 2 * BK - 1) // (2 * BK)
    half = grid * BK

    def body(src, out):
        lo = jnp.transpose(src[:, pl.ds(0, BK)])
        hi = jnp.transpose(src[:, pl.ds(BK, BK)])
        out[...] = jnp.concatenate([lo, hi], axis=1)

    return pl.pallas_call(
        body,
        grid=(grid,),
        in_specs=[pl.BlockSpec((d, 2 * BK), lambda i: (0, i))],
        out_specs=pl.BlockSpec((BK, 2 * d), lambda i: (i, 0)),
        out_shape=jax.ShapeDtypeStruct((half, 2 * d), jnp.float32),
    )(table.T)


@functools.partial(
    pl.kernel,
    mesh=_mesh,
    compiler_params=pltpu.CompilerParams(use_tc_tiling_on_sc=True),
    out_type=jax.ShapeDtypeStruct((B,), jnp.float32),
    scratch_types=[
        pltpu.VMEM((NCH, CH), jnp.int32),
        pltpu.VMEM((NCH, CH), jnp.int32),
        pltpu.VMEM((NCH, CH), jnp.int32),
        pltpu.VMEM((NCH, CH), jnp.int32),
        pltpu.VMEM((CH, 2 * D), jnp.float32),
        pltpu.VMEM((CH, 2 * D), jnp.float32),
        pltpu.VMEM((BPW,), jnp.float32),
        pltpu.VMEM((BPW,), jnp.float32),
        pltpu.VMEM((BPW,), jnp.float32),
        pltpu.SemaphoreType.DMA,
    ],
)
def _rec_sc(uid_hbm, fid_hbm, ut_hbm, ft_hbm, ub_hbm, fb_hbm, out_hbm,
            uid_o, fid_o, uid_s, fid_s, urows, frows, ubias, fbias, sums, sem):
    wid = lax.axis_index("s") * NC + lax.axis_index("c")
    base = wid * BPW

    pltpu.sync_copy(uid_hbm.at[wid], uid_o)
    pltpu.sync_copy(fid_hbm.at[wid], fid_o)

    bias_copies = []
    for j in range(NCH):
        sl = pl.ds(j * CH, CH)
        bias_copies.append(
            pltpu.async_copy(ub_hbm.at[uid_o.at[j]], ubias.at[sl], sem))
        bias_copies.append(
            pltpu.async_copy(fb_hbm.at[fid_o.at[j]], fbias.at[sl], sem))

    for j in range(NCH):
        for k in range(0, CH, L):
            s = pl.ds(k, L)
            for src, dst in ((uid_o, uid_s), (fid_o, fid_s)):
                x = src[j, s]
                dst[j, s] = (
                    lax.shift_left(lax.shift_right_logical(x, 10), 9)
                    | (x & 511))

    lane_iota = lax.iota(jnp.int32, L)

    for j in range(NCH):
        cu = pltpu.async_copy(ut_hbm.at[uid_s.at[j]], urows, sem)
        cf = pltpu.async_copy(ft_hbm.at[fid_s.at[j]], frows, sem)
        cu.wait()
        cf.wait()

        def grp_body(g, carry, j=j):
            rowsums = jnp.zeros((L,), jnp.float32)
            pu = (lax.shift_right_logical(uid_o[j, pl.ds(g * L, L)], 9) & 1) * D
            pf = (lax.shift_right_logical(fid_o[j, pl.ds(g * L, L)], 9) & 1) * D
            for k in range(L):
                r = g * L + k
                uoff = pu[k]
                foff = pf[k]
                acc = (urows[r, pl.ds(uoff, L)] * frows[r, pl.ds(foff, L)])
                for q in range(1, D // L):
                    acc = acc + (urows[r, pl.ds(uoff + q * L, L)]
                                 * frows[r, pl.ds(foff + q * L, L)])
                for sh in (8, 4, 2, 1):
                    acc = acc + _perm(acc, lane_iota ^ sh)
                rowsums = jnp.where(lane_iota == k, acc, rowsums)
            sums[pl.ds(j * CH + g * L, L)] = rowsums
            return carry

        lax.fori_loop(0, CH // L, grp_body, 0)

    for c in bias_copies:
        c.wait()

    def bias_body(i, carry):
        s = pl.ds(i * L, L)
        sums[s] = sums[s] + ubias[s] + fbias[s]
        return carry

    lax.fori_loop(0, BPW // L, bias_body, 0)

    pltpu.sync_copy(sums, out_hbm.at[pl.ds(base, BPW)])


def kernel(user_id, film_id, user_table, film_table, user_bias_table, film_bias_table):
    uid3d = user_id.astype(jnp.int32).reshape(NW, NCH, CH)
    fid3d = film_id.astype(jnp.int32).reshape(NW, NCH, CH)
    ut2 = _pack_rows(user_table)
    ft2 = _pack_rows(film_table)
    ub = user_bias_table.reshape(-1)
    fb = film_bias_table.reshape(-1)
    return _rec_sc(uid3d, fid3d, ut2, ft2, ub, fb)


# speedup vs baseline: 1.9134x; 1.9134x over previous
import functools

import jax
import jax.numpy as jnp
from jax import lax
from jax.experimental import pallas as pl
from jax.experimental.pallas import tpu as pltpu
from jax.experimental.pallas import tpu_sc as plsc

NC = 2
NS = 16
NW = NC * NS
B = 16384
D = 64
L = 16
BPW = B // NW
CH = 128
NCH = BPW // CH
BK = 512
W = 4

_mesh = plsc.VectorSubcoreMesh(core_axis_name="c", subcore_axis_name="s")

_GATHER_DN = lax.GatherDimensionNumbers(
    offset_dims=(), collapsed_slice_dims=(0,), start_index_map=(0,))


def _perm(v, idx):
    return lax.gather(v, idx[:, None], _GATHER_DN, slice_sizes=(1,),
                      mode=lax.GatherScatterMode.PROMISE_IN_BOUNDS)


def _pack_rows(table):
    v, d = table.shape
    win = 2 * BK * W
    grid = (v + win - 1) // win

    def body(src, out):
        ident = (lax.broadcasted_iota(jnp.int32, (d, d), 0)
                 == lax.broadcasted_iota(jnp.int32, (d, d), 1)
                 ).astype(jnp.float32)
        y = lax.dot_general(src[...], ident, (((0,), (0,)), ((), ())),
                            preferred_element_type=jnp.float32)
        for w in range(W):
            lo = y[2 * w * BK:2 * w * BK + BK]
            hi = y[2 * w * BK + BK:2 * (w + 1) * BK]
            out[pl.ds(w * BK, BK), :] = jnp.concatenate([lo, hi], axis=1)

    return pl.pallas_call(
        body,
        grid=(grid,),
        in_specs=[pl.BlockSpec((d, win), lambda i: (0, i))],
        out_specs=pl.BlockSpec((W * BK, 2 * d), lambda i: (i, 0)),
        out_shape=jax.ShapeDtypeStruct((grid * W * BK, 2 * d), jnp.float32),
    )(table.T)


@functools.partial(
    pl.kernel,
    mesh=_mesh,
    compiler_params=pltpu.CompilerParams(use_tc_tiling_on_sc=True),
    out_type=jax.ShapeDtypeStruct((B,), jnp.float32),
    scratch_types=[
        pltpu.VMEM((NCH, CH), jnp.int32),
        pltpu.VMEM((NCH, CH), jnp.int32),
        pltpu.VMEM((NCH, CH), jnp.int32),
        pltpu.VMEM((NCH, CH), jnp.int32),
        pltpu.VMEM((CH, 2 * D), jnp.float32),
        pltpu.VMEM((CH, 2 * D), jnp.float32),
        pltpu.VMEM((BPW,), jnp.float32),
        pltpu.VMEM((BPW,), jnp.float32),
        pltpu.VMEM((BPW,), jnp.float32),
        pltpu.SemaphoreType.DMA,
    ],
)
def _rec_sc(uid_hbm, fid_hbm, ut_hbm, ft_hbm, ub_hbm, fb_hbm, out_hbm,
            uid_o, fid_o, uid_s, fid_s, urows, frows, ubias, fbias, sums, sem):
    wid = lax.axis_index("s") * NC + lax.axis_index("c")
    base = wid * BPW

    pltpu.sync_copy(uid_hbm.at[wid], uid_o)
    pltpu.sync_copy(fid_hbm.at[wid], fid_o)

    bias_copies = []
    for j in range(NCH):
        sl = pl.ds(j * CH, CH)
        bias_copies.append(
            pltpu.async_copy(ub_hbm.at[uid_o.at[j]], ubias.at[sl], sem))
        bias_copies.append(
            pltpu.async_copy(fb_hbm.at[fid_o.at[j]], fbias.at[sl], sem))

    for j in range(NCH):
        for k in range(0, CH, L):
            s = pl.ds(k, L)
            for src, dst in ((uid_o, uid_s), (fid_o, fid_s)):
                x = src[j, s]
                dst[j, s] = (
                    lax.shift_left(lax.shift_right_logical(x, 10), 9)
                    | (x & 511))

    lane_iota = lax.iota(jnp.int32, L)

    for j in range(NCH):
        cu = pltpu.async_copy(ut_hbm.at[uid_s.at[j]], urows, sem)
        cf = pltpu.async_copy(ft_hbm.at[fid_s.at[j]], frows, sem)
        cu.wait()
        cf.wait()

        def grp_body(g, carry, j=j):
            rowsums = jnp.zeros((L,), jnp.float32)
            pu = (lax.shift_right_logical(uid_o[j, pl.ds(g * L, L)], 9) & 1) * D
            pf = (lax.shift_right_logical(fid_o[j, pl.ds(g * L, L)], 9) & 1) * D
            for k in range(L):
                r = g * L + k
                uoff = pu[k]
                foff = pf[k]
                acc = (urows[r, pl.ds(uoff, L)] * frows[r, pl.ds(foff, L)])
                for q in range(1, D // L):
                    acc = acc + (urows[r, pl.ds(uoff + q * L, L)]
                                 * frows[r, pl.ds(foff + q * L, L)])
                for sh in (8, 4, 2, 1):
                    acc = acc + _perm(acc, lane_iota ^ sh)
                rowsums = jnp.where(lane_iota == k, acc, rowsums)
            sums[pl.ds(j * CH + g * L, L)] = rowsums
            return carry

        lax.fori_loop(0, CH // L, grp_body, 0)

    for c in bias_copies:
        c.wait()

    def bias_body(i, carry):
        s = pl.ds(i * L, L)
        sums[s] = sums[s] + ubias[s] + fbias[s]
        return carry

    lax.fori_loop(0, BPW // L, bias_body, 0)

    pltpu.sync_copy(sums, out_hbm.at[pl.ds(base, BPW)])


def kernel(user_id, film_id, user_table, film_table, user_bias_table, film_bias_table):
    uid3d = user_id.astype(jnp.int32).reshape(NW, NCH, CH)
    fid3d = film_id.astype(jnp.int32).reshape(NW, NCH, CH)
    ut2 = _pack_rows(user_table)
    ft2 = _pack_rows(film_table)
    ub = user_bias_table.reshape(-1)
    fb = film_bias_table.reshape(-1)
    return _rec_sc(uid3d, fid3d, ut2, ft2, ub, fb)


# speedup vs baseline: 2.3074x; 1.2059x over previous
import functools

import jax
import jax.numpy as jnp
from jax import lax
from jax.experimental import pallas as pl
from jax.experimental.pallas import tpu as pltpu
from jax.experimental.pallas import tpu_sc as plsc

NC = 2
NS = 16
NW = NC * NS
B = 16384
D = 64
L = 16
BPW = B // NW
CH = 128
NCH = BPW // CH
BK = 512
W = 8

_mesh = plsc.VectorSubcoreMesh(core_axis_name="c", subcore_axis_name="s")

_GATHER_DN = lax.GatherDimensionNumbers(
    offset_dims=(), collapsed_slice_dims=(0,), start_index_map=(0,))


def _perm(v, idx):
    return lax.gather(v, idx[:, None], _GATHER_DN, slice_sizes=(1,),
                      mode=lax.GatherScatterMode.PROMISE_IN_BOUNDS)


def _pack_rows(table):
    v, d = table.shape
    win = 2 * BK * W
    grid = (v + win - 1) // win

    def body(src, out):
        for w in range(W):
            lo = jnp.transpose(src[:, pl.ds(2 * w * BK, BK)])
            hi = jnp.transpose(src[:, pl.ds(2 * w * BK + BK, BK)])
            out[pl.ds(w * BK, BK), :] = jnp.concatenate([lo, hi], axis=1)

    return pl.pallas_call(
        body,
        grid=(grid,),
        in_specs=[pl.BlockSpec((d, win), lambda i: (0, i))],
        out_specs=pl.BlockSpec((W * BK, 2 * d), lambda i: (i, 0)),
        out_shape=jax.ShapeDtypeStruct((grid * W * BK, 2 * d), jnp.float32),
    )(table.T)


@functools.partial(
    pl.kernel,
    mesh=_mesh,
    compiler_params=pltpu.CompilerParams(use_tc_tiling_on_sc=True),
    out_type=jax.ShapeDtypeStruct((B,), jnp.float32),
    scratch_types=[
        pltpu.VMEM((NCH, CH), jnp.int32),
        pltpu.VMEM((NCH, CH), jnp.int32),
        pltpu.VMEM((NCH, CH), jnp.int32),
        pltpu.VMEM((NCH, CH), jnp.int32),
        pltpu.VMEM((CH, 2 * D), jnp.float32),
        pltpu.VMEM((CH, 2 * D), jnp.float32),
        pltpu.VMEM((BPW,), jnp.float32),
        pltpu.VMEM((BPW,), jnp.float32),
        pltpu.VMEM((BPW,), jnp.float32),
        pltpu.SemaphoreType.DMA,
    ],
)
def _rec_sc(uid_hbm, fid_hbm, ut_hbm, ft_hbm, ub_hbm, fb_hbm, out_hbm,
            uid_o, fid_o, uid_s, fid_s, urows, frows, ubias, fbias, sums, sem):
    wid = lax.axis_index("s") * NC + lax.axis_index("c")
    base = wid * BPW

    pltpu.sync_copy(uid_hbm.at[wid], uid_o)
    pltpu.sync_copy(fid_hbm.at[wid], fid_o)

    bias_copies = []
    for j in range(NCH):
        sl = pl.ds(j * CH, CH)
        bias_copies.append(
            pltpu.async_copy(ub_hbm.at[uid_o.at[j]], ubias.at[sl], sem))
        bias_copies.append(
            pltpu.async_copy(fb_hbm.at[fid_o.at[j]], fbias.at[sl], sem))

    for j in range(NCH):
        for k in range(0, CH, L):
            s = pl.ds(k, L)
            for src, dst in ((uid_o, uid_s), (fid_o, fid_s)):
                x = src[j, s]
                dst[j, s] = (
                    lax.shift_left(lax.shift_right_logical(x, 10), 9)
                    | (x & 511))

    lane_iota = lax.iota(jnp.int32, L)

    for j in range(NCH):
        cu = pltpu.async_copy(ut_hbm.at[uid_s.at[j]], urows, sem)
        cf = pltpu.async_copy(ft_hbm.at[fid_s.at[j]], frows, sem)
        cu.wait()
        cf.wait()

        def grp_body(g, carry, j=j):
            rowsums = jnp.zeros((L,), jnp.float32)
            pu = (lax.shift_right_logical(uid_o[j, pl.ds(g * L, L)], 9) & 1) * D
            pf = (lax.shift_right_logical(fid_o[j, pl.ds(g * L, L)], 9) & 1) * D
            for k in range(L):
                r = g * L + k
                uoff = pu[k]
                foff = pf[k]
                acc = (urows[r, pl.ds(uoff, L)] * frows[r, pl.ds(foff, L)])
                for q in range(1, D // L):
                    acc = acc + (urows[r, pl.ds(uoff + q * L, L)]
                                 * frows[r, pl.ds(foff + q * L, L)])
                for sh in (8, 4, 2, 1):
                    acc = acc + _perm(acc, lane_iota ^ sh)
                rowsums = jnp.where(lane_iota == k, acc, rowsums)
            sums[pl.ds(j * CH + g * L, L)] = rowsums
            return carry

        lax.fori_loop(0, CH // L, grp_body, 0)

    for c in bias_copies:
        c.wait()

    def bias_body(i, carry):
        s = pl.ds(i * L, L)
        sums[s] = sums[s] + ubias[s] + fbias[s]
        return carry

    lax.fori_loop(0, BPW // L, bias_body, 0)

    pltpu.sync_copy(sums, out_hbm.at[pl.ds(base, BPW)])


def kernel(user_id, film_id, user_table, film_table, user_bias_table, film_bias_table):
    uid3d = user_id.astype(jnp.int32).reshape(NW, NCH, CH)
    fid3d = film_id.astype(jnp.int32).reshape(NW, NCH, CH)
    ut2 = _pack_rows(user_table)
    ft2 = _pack_rows(film_table)
    ub = user_bias_table.reshape(-1)
    fb = film_bias_table.reshape(-1)
    return _rec_sc(uid3d, fid3d, ut2, ft2, ub, fb)


# speedup vs baseline: 2.5377x; 1.0998x over previous
import functools

import jax
import jax.numpy as jnp
from jax import lax
from jax.experimental import pallas as pl
from jax.experimental.pallas import tpu as pltpu
from jax.experimental.pallas import tpu_sc as plsc

NC = 2
NS = 16
NW = NC * NS
B = 16384
D = 64
L = 16
BPW = B // NW
CH = 128
NCH = BPW // CH
BK = 512
W = 16

_mesh = plsc.VectorSubcoreMesh(core_axis_name="c", subcore_axis_name="s")

_GATHER_DN = lax.GatherDimensionNumbers(
    offset_dims=(), collapsed_slice_dims=(0,), start_index_map=(0,))


def _perm(v, idx):
    return lax.gather(v, idx[:, None], _GATHER_DN, slice_sizes=(1,),
                      mode=lax.GatherScatterMode.PROMISE_IN_BOUNDS)


def _pack_rows(table):
    v, d = table.shape
    win = 2 * BK * W
    grid = (v + win - 1) // win

    def body(src, out):
        for w in range(W):
            lo = jnp.transpose(src[:, pl.ds(2 * w * BK, BK)])
            hi = jnp.transpose(src[:, pl.ds(2 * w * BK + BK, BK)])
            out[pl.ds(w * BK, BK), :] = jnp.concatenate([lo, hi], axis=1)

    return pl.pallas_call(
        body,
        grid=(grid,),
        in_specs=[pl.BlockSpec((d, win), lambda i: (0, i))],
        out_specs=pl.BlockSpec((W * BK, 2 * d), lambda i: (i, 0)),
        out_shape=jax.ShapeDtypeStruct((grid * W * BK, 2 * d), jnp.float32),
    )(table.T)


@functools.partial(
    pl.kernel,
    mesh=_mesh,
    compiler_params=pltpu.CompilerParams(use_tc_tiling_on_sc=True),
    out_type=jax.ShapeDtypeStruct((B,), jnp.float32),
    scratch_types=[
        pltpu.VMEM((NCH, CH), jnp.int32),
        pltpu.VMEM((NCH, CH), jnp.int32),
        pltpu.VMEM((NCH, CH), jnp.int32),
        pltpu.VMEM((NCH, CH), jnp.int32),
        pltpu.VMEM((CH, 2 * D), jnp.float32),
        pltpu.VMEM((CH, 2 * D), jnp.float32),
        pltpu.VMEM((BPW,), jnp.float32),
        pltpu.VMEM((BPW,), jnp.float32),
        pltpu.VMEM((BPW,), jnp.float32),
        pltpu.SemaphoreType.DMA,
    ],
)
def _rec_sc(uid_hbm, fid_hbm, ut_hbm, ft_hbm, ub_hbm, fb_hbm, out_hbm,
            uid_o, fid_o, uid_s, fid_s, urows, frows, ubias, fbias, sums, sem):
    wid = lax.axis_index("s") * NC + lax.axis_index("c")
    base = wid * BPW

    pltpu.sync_copy(uid_hbm.at[wid], uid_o)
    pltpu.sync_copy(fid_hbm.at[wid], fid_o)

    bias_copies = []
    for j in range(NCH):
        sl = pl.ds(j * CH, CH)
        bias_copies.append(
            pltpu.async_copy(ub_hbm.at[uid_o.at[j]], ubias.at[sl], sem))
        bias_copies.append(
            pltpu.async_copy(fb_hbm.at[fid_o.at[j]], fbias.at[sl], sem))

    for j in range(NCH):
        for k in range(0, CH, L):
            s = pl.ds(k, L)
            for src, dst in ((uid_o, uid_s), (fid_o, fid_s)):
                x = src[j, s]
                dst[j, s] = (
                    lax.shift_left(lax.shift_right_logical(x, 10), 9)
                    | (x & 511))

    lane_iota = lax.iota(jnp.int32, L)

    for j in range(NCH):
        cu = pltpu.async_copy(ut_hbm.at[uid_s.at[j]], urows, sem)
        cf = pltpu.async_copy(ft_hbm.at[fid_s.at[j]], frows, sem)
        cu.wait()
        cf.wait()

        def grp_body(g, carry, j=j):
            rowsums = jnp.zeros((L,), jnp.float32)
            pu = (lax.shift_right_logical(uid_o[j, pl.ds(g * L, L)], 9) & 1) * D
            pf = (lax.shift_right_logical(fid_o[j, pl.ds(g * L, L)], 9) & 1) * D
            for k in range(L):
                r = g * L + k
                uoff = pu[k]
                foff = pf[k]
                acc = (urows[r, pl.ds(uoff, L)] * frows[r, pl.ds(foff, L)])
                for q in range(1, D // L):
                    acc = acc + (urows[r, pl.ds(uoff + q * L, L)]
                                 * frows[r, pl.ds(foff + q * L, L)])
                for sh in (8, 4, 2, 1):
                    acc = acc + _perm(acc, lane_iota ^ sh)
                rowsums = jnp.where(lane_iota == k, acc, rowsums)
            sums[pl.ds(j * CH + g * L, L)] = rowsums
            return carry

        lax.fori_loop(0, CH // L, grp_body, 0)

    for c in bias_copies:
        c.wait()

    def bias_body(i, carry):
        s = pl.ds(i * L, L)
        sums[s] = sums[s] + ubias[s] + fbias[s]
        return carry

    lax.fori_loop(0, BPW // L, bias_body, 0)

    pltpu.sync_copy(sums, out_hbm.at[pl.ds(base, BPW)])


def kernel(user_id, film_id, user_table, film_table, user_bias_table, film_bias_table):
    uid3d = user_id.astype(jnp.int32).reshape(NW, NCH, CH)
    fid3d = film_id.astype(jnp.int32).reshape(NW, NCH, CH)
    ut2 = _pack_rows(user_table)
    ft2 = _pack_rows(film_table)
    ub = user_bias_table.reshape(-1)
    fb = film_bias_table.reshape(-1)
    return _rec_sc(uid3d, fid3d, ut2, ft2, ub, fb)


# speedup vs baseline: 2.6157x; 1.0308x over previous
import functools

import jax
import jax.numpy as jnp
from jax import lax
from jax.experimental import pallas as pl
from jax.experimental.pallas import tpu as pltpu
from jax.experimental.pallas import tpu_sc as plsc

NC = 2
NS = 16
NW = NC * NS
B = 16384
D = 64
L = 16
BPW = B // NW
CH = 128
NCH = BPW // CH
BK = 512
W = 32

_mesh = plsc.VectorSubcoreMesh(core_axis_name="c", subcore_axis_name="s")

_GATHER_DN = lax.GatherDimensionNumbers(
    offset_dims=(), collapsed_slice_dims=(0,), start_index_map=(0,))


def _perm(v, idx):
    return lax.gather(v, idx[:, None], _GATHER_DN, slice_sizes=(1,),
                      mode=lax.GatherScatterMode.PROMISE_IN_BOUNDS)


def _pack_rows(table):
    v, d = table.shape
    win = 2 * BK * W
    grid = (v + win - 1) // win

    def body(src, out):
        for w in range(W):
            lo = jnp.transpose(src[:, pl.ds(2 * w * BK, BK)])
            hi = jnp.transpose(src[:, pl.ds(2 * w * BK + BK, BK)])
            out[pl.ds(w * BK, BK), :] = jnp.concatenate([lo, hi], axis=1)

    return pl.pallas_call(
        body,
        grid=(grid,),
        in_specs=[pl.BlockSpec((d, win), lambda i: (0, i))],
        out_specs=pl.BlockSpec((W * BK, 2 * d), lambda i: (i, 0)),
        out_shape=jax.ShapeDtypeStruct((grid * W * BK, 2 * d), jnp.float32),
    )(table.T)


@functools.partial(
    pl.kernel,
    mesh=_mesh,
    compiler_params=pltpu.CompilerParams(use_tc_tiling_on_sc=True),
    out_type=jax.ShapeDtypeStruct((B,), jnp.float32),
    scratch_types=[
        pltpu.VMEM((NCH, CH), jnp.int32),
        pltpu.VMEM((NCH, CH), jnp.int32),
        pltpu.VMEM((NCH, CH), jnp.int32),
        pltpu.VMEM((NCH, CH), jnp.int32),
        pltpu.VMEM((CH, 2 * D), jnp.float32),
        pltpu.VMEM((CH, 2 * D), jnp.float32),
        pltpu.VMEM((BPW,), jnp.float32),
        pltpu.VMEM((BPW,), jnp.float32),
        pltpu.VMEM((BPW,), jnp.float32),
        pltpu.SemaphoreType.DMA,
    ],
)
def _rec_sc(uid_hbm, fid_hbm, ut_hbm, ft_hbm, ub_hbm, fb_hbm, out_hbm,
            uid_o, fid_o, uid_s, fid_s, urows, frows, ubias, fbias, sums, sem):
    wid = lax.axis_index("s") * NC + lax.axis_index("c")
    base = wid * BPW

    pltpu.sync_copy(uid_hbm.at[wid], uid_o)
    pltpu.sync_copy(fid_hbm.at[wid], fid_o)

    bias_copies = []
    for j in range(NCH):
        sl = pl.ds(j * CH, CH)
        bias_copies.append(
            pltpu.async_copy(ub_hbm.at[uid_o.at[j]], ubias.at[sl], sem))
        bias_copies.append(
            pltpu.async_copy(fb_hbm.at[fid_o.at[j]], fbias.at[sl], sem))

    for j in range(NCH):
        for k in range(0, CH, L):
            s = pl.ds(k, L)
            for src, dst in ((uid_o, uid_s), (fid_o, fid_s)):
                x = src[j, s]
                dst[j, s] = (
                    lax.shift_left(lax.shift_right_logical(x, 10), 9)
                    | (x & 511))

    lane_iota = lax.iota(jnp.int32, L)

    for j in range(NCH):
        cu = pltpu.async_copy(ut_hbm.at[uid_s.at[j]], urows, sem)
        cf = pltpu.async_copy(ft_hbm.at[fid_s.at[j]], frows, sem)
        cu.wait()
        cf.wait()

        def grp_body(g, carry, j=j):
            rowsums = jnp.zeros((L,), jnp.float32)
            pu = (lax.shift_right_logical(uid_o[j, pl.ds(g * L, L)], 9) & 1) * D
            pf = (lax.shift_right_logical(fid_o[j, pl.ds(g * L, L)], 9) & 1) * D
            for k in range(L):
                r = g * L + k
                uoff = pu[k]
                foff = pf[k]
                acc = (urows[r, pl.ds(uoff, L)] * frows[r, pl.ds(foff, L)])
                for q in range(1, D // L):
                    acc = acc + (urows[r, pl.ds(uoff + q * L, L)]
                                 * frows[r, pl.ds(foff + q * L, L)])
                for sh in (8, 4, 2, 1):
                    acc = acc + _perm(acc, lane_iota ^ sh)
                rowsums = jnp.where(lane_iota == k, acc, rowsums)
            sums[pl.ds(j * CH + g * L, L)] = rowsums
            return carry

        lax.fori_loop(0, CH // L, grp_body, 0)

    for c in bias_copies:
        c.wait()

    def bias_body(i, carry):
        s = pl.ds(i * L, L)
        sums[s] = sums[s] + ubias[s] + fbias[s]
        return carry

    lax.fori_loop(0, BPW // L, bias_body, 0)

    pltpu.sync_copy(sums, out_hbm.at[pl.ds(base, BPW)])


def kernel(user_id, film_id, user_table, film_table, user_bias_table, film_bias_table):
    uid3d = user_id.astype(jnp.int32).reshape(NW, NCH, CH)
    fid3d = film_id.astype(jnp.int32).reshape(NW, NCH, CH)
    ut2 = _pack_rows(user_table)
    ft2 = _pack_rows(film_table)
    ub = user_bias_table.reshape(-1)
    fb = film_bias_table.reshape(-1)
    return _rec_sc(uid3d, fid3d, ut2, ft2, ub, fb)


# speedup vs baseline: 2.6435x; 1.0106x over previous
import functools

import jax
import jax.numpy as jnp
from jax import lax
from jax.experimental import pallas as pl
from jax.experimental.pallas import tpu as pltpu
from jax.experimental.pallas import tpu_sc as plsc

NC = 2
NS = 16
NW = NC * NS
B = 16384
D = 64
L = 16
BPW = B // NW
CH = 128
NCH = BPW // CH
BK = 512
W = 32

_mesh = plsc.VectorSubcoreMesh(core_axis_name="c", subcore_axis_name="s")

_GATHER_DN = lax.GatherDimensionNumbers(
    offset_dims=(), collapsed_slice_dims=(0,), start_index_map=(0,))


def _perm(v, idx):
    return lax.gather(v, idx[:, None], _GATHER_DN, slice_sizes=(1,),
                      mode=lax.GatherScatterMode.PROMISE_IN_BOUNDS)


def _pack_rows(table):
    v, d = table.shape
    win = 2 * BK * W
    grid = (v + win - 1) // win

    def body(src, out):
        for w in range(W):
            lo = jnp.transpose(src[:, pl.ds(2 * w * BK, BK)])
            hi = jnp.transpose(src[:, pl.ds(2 * w * BK + BK, BK)])
            out[pl.ds(w * BK, BK), :] = jnp.concatenate([lo, hi], axis=1)

    return pl.pallas_call(
        body,
        grid=(grid,),
        in_specs=[pl.BlockSpec((d, win), lambda i: (0, i))],
        out_specs=pl.BlockSpec((W * BK, 2 * d), lambda i: (i, 0)),
        out_shape=jax.ShapeDtypeStruct((grid * W * BK, 2 * d), jnp.float32),
    )(table.T)


@functools.partial(
    pl.kernel,
    mesh=_mesh,
    compiler_params=pltpu.CompilerParams(use_tc_tiling_on_sc=True),
    out_type=jax.ShapeDtypeStruct((B,), jnp.float32),
    scratch_types=[
        pltpu.VMEM((NCH, CH), jnp.int32),
        pltpu.VMEM((NCH, CH), jnp.int32),
        pltpu.VMEM((NCH, CH), jnp.int32),
        pltpu.VMEM((NCH, CH), jnp.int32),
        pltpu.VMEM((CH, 2 * D), jnp.float32),
        pltpu.VMEM((CH, 2 * D), jnp.float32),
        pltpu.VMEM((CH, 2 * D), jnp.float32),
        pltpu.VMEM((CH, 2 * D), jnp.float32),
        pltpu.VMEM((BPW,), jnp.float32),
        pltpu.VMEM((BPW,), jnp.float32),
        pltpu.VMEM((BPW,), jnp.float32),
        pltpu.SemaphoreType.DMA,
        pltpu.SemaphoreType.DMA,
        pltpu.SemaphoreType.DMA,
    ],
)
def _rec_sc(uid_hbm, fid_hbm, ut_hbm, ft_hbm, ub_hbm, fb_hbm, out_hbm,
            uid_o, fid_o, uid_s, fid_s, urows0, frows0, urows1, frows1,
            ubias, fbias, sums, sem_b, sem0, sem1):
    wid = lax.axis_index("s") * NC + lax.axis_index("c")
    base = wid * BPW

    pltpu.sync_copy(uid_hbm.at[wid], uid_o)
    pltpu.sync_copy(fid_hbm.at[wid], fid_o)

    bias_copies = []
    for j in range(NCH):
        sl = pl.ds(j * CH, CH)
        bias_copies.append(
            pltpu.async_copy(ub_hbm.at[uid_o.at[j]], ubias.at[sl], sem_b))
        bias_copies.append(
            pltpu.async_copy(fb_hbm.at[fid_o.at[j]], fbias.at[sl], sem_b))

    for j in range(NCH):
        for k in range(0, CH, L):
            s = pl.ds(k, L)
            for src, dst in ((uid_o, uid_s), (fid_o, fid_s)):
                x = src[j, s]
                dst[j, s] = (
                    lax.shift_left(lax.shift_right_logical(x, 10), 9)
                    | (x & 511))

    lane_iota = lax.iota(jnp.int32, L)

    bufs = [(urows0, frows0, sem0), (urows1, frows1, sem1)]
    pend = [None] * NCH

    def issue(j):
        ur, fr, sm = bufs[j % 2]
        pend[j] = (pltpu.async_copy(ut_hbm.at[uid_s.at[j]], ur, sm),
                   pltpu.async_copy(ft_hbm.at[fid_s.at[j]], fr, sm))

    issue(0)
    for j in range(NCH):
        if j + 1 < NCH:
            issue(j + 1)
        cu, cf = pend[j]
        cu.wait()
        cf.wait()
        urows, frows, _ = bufs[j % 2]

        def grp_body(g, carry, j=j, urows=urows, frows=frows):
            rowsums = jnp.zeros((L,), jnp.float32)
            pu = (lax.shift_right_logical(uid_o[j, pl.ds(g * L, L)], 9) & 1) * D
            pf = (lax.shift_right_logical(fid_o[j, pl.ds(g * L, L)], 9) & 1) * D
            for k in range(L):
                r = g * L + k
                uoff = pu[k]
                foff = pf[k]
                acc = (urows[r, pl.ds(uoff, L)] * frows[r, pl.ds(foff, L)])
                for q in range(1, D // L):
                    acc = acc + (urows[r, pl.ds(uoff + q * L, L)]
                                 * frows[r, pl.ds(foff + q * L, L)])
                for sh in (8, 4, 2, 1):
                    acc = acc + _perm(acc, lane_iota ^ sh)
                rowsums = jnp.where(lane_iota == k, acc, rowsums)
            sums[pl.ds(j * CH + g * L, L)] = rowsums
            return carry

        lax.fori_loop(0, CH // L, grp_body, 0)

    for c in bias_copies:
        c.wait()

    def bias_body(i, carry):
        s = pl.ds(i * L, L)
        sums[s] = sums[s] + ubias[s] + fbias[s]
        return carry

    lax.fori_loop(0, BPW // L, bias_body, 0)

    pltpu.sync_copy(sums, out_hbm.at[pl.ds(base, BPW)])


def kernel(user_id, film_id, user_table, film_table, user_bias_table, film_bias_table):
    uid3d = user_id.astype(jnp.int32).reshape(NW, NCH, CH)
    fid3d = film_id.astype(jnp.int32).reshape(NW, NCH, CH)
    ut2 = _pack_rows(user_table)
    ft2 = _pack_rows(film_table)
    ub = user_bias_table.reshape(-1)
    fb = film_bias_table.reshape(-1)
    return _rec_sc(uid3d, fid3d, ut2, ft2, ub, fb)
